# baseline (speedup 1.0000x reference)
import functools

import jax
import jax.numpy as jnp
import numpy as np
from jax import lax
from jax.experimental import pallas as pl
from jax.experimental.pallas import tpu as pltpu
from jax.experimental.pallas import tpu_sc as plsc

U32 = jnp.uint32
I32 = jnp.int32
F32 = jnp.float32
_SCALE15625 = np.float32((1.0 - 2.0**-21) / 15625.0)


def _c(v):
    return U32(v)


def _mul64c(lo, hi, c):
    c = _c(c)
    l0 = lo & _c(0xFFFF)
    l1 = lo >> _c(16)
    p0 = l0 * c
    p1 = l1 * c
    nlo = (p1 << _c(16)) + p0
    ch = (p1 + (p0 >> _c(16))) >> _c(16)
    h0 = hi & _c(0xFFFF)
    h1 = hi >> _c(16)
    nhi = ((h1 * c) << _c(16)) + h0 * c + ch
    return nlo, nhi


def _xor_shr(lo, hi, s):
    slo = (lo >> _c(s)) | (hi << _c(32 - s))
    shi = hi >> _c(s)
    return lo ^ slo, hi ^ shi


def _hash_i32(v):
    return plsc.bitcast(_hash_vec(plsc.bitcast(v, U32)), I32)


def _hash_vec(x):
    blo = x << _c(21)
    bhi = x >> _c(11)
    alo = ~x
    lo = alo + blo
    carry = jnp.where(lo < alo, _c(1), _c(0))
    hi = bhi + carry + _c(0xFFFFFFFF)
    lo, hi = _xor_shr(lo, hi, 24)
    lo, hi = _mul64c(lo, hi, 265)
    lo, hi = _xor_shr(lo, hi, 14)
    lo, hi = _mul64c(lo, hi, 21)
    lo, hi = _xor_shr(lo, hi, 28)
    b0m = lo << _c(31)
    nlo = lo ^ b0m
    c7 = (lo & b0m) >> _c(31)
    shi = (hi << _c(31)) | (lo >> _c(1))
    hi = hi + shi + c7
    lo = nlo
    r0 = lo & _c(63)
    qlo = (lo >> _c(6)) | (hi << _c(26))
    qhi = hi >> _c(6)
    w0 = qlo & _c(0xFFFF)
    w1 = qlo >> _c(16)
    w2 = qhi & _c(0xFFFF)
    w3 = qhi >> _c(16)
    s = w0 + w1 * _c(3036) + w2 * _c(14171) + w3 * _c(7531)
    s = s.astype(I32)
    q = (s.astype(F32) * _SCALE15625).astype(I32)
    r = s - q * I32(15625)
    t = r - I32(15625)
    r = t + ((t >> I32(31)) & I32(15625))
    return (r.astype(U32) << _c(6)) | r0


def _make_sc_call(rows, cols):
    n = rows * cols
    info = plsc.get_sparse_core_info()
    nc, ns = info.num_cores, info.num_subcores
    nw = nc * ns
    per_w = n // nw
    groups = 4
    rows_per_w = rows // nw
    grows = rows_per_w // groups
    chunk = grows * cols
    assert per_w * nw == n and grows * groups == rows_per_w
    assert chunk % 16 == 0
    mesh = plsc.VectorSubcoreMesh(core_axis_name="c", subcore_axis_name="s")

    @functools.partial(
        pl.kernel,
        mesh=mesh,
        out_type=jax.ShapeDtypeStruct((rows, cols), jnp.int32),
        scratch_types=[pltpu.VMEM((grows, cols), jnp.int32)] * groups
        + [pltpu.SemaphoreType.DMA] * (2 * groups),
    )
    def sc_hash(x_hbm, out_hbm, b0, b1, b2, b3, *sems):
        bufs = [b0, b1, b2, b3]
        in_sems = sems[:groups]
        out_sems = sems[groups:]
        wid = lax.axis_index("s") * I32(nc) + lax.axis_index("c")
        rbase = wid * I32(rows_per_w)

        def issue_in(g):
            roff = pl.multiple_of(rbase + I32(g * grows), grows)
            return pltpu.async_copy(
                x_hbm.at[pl.ds(roff, grows)], bufs[g], in_sems[g])

        def issue_out(g):
            roff = pl.multiple_of(rbase + I32(g * grows), grows)
            return pltpu.async_copy(
                bufs[g], out_hbm.at[pl.ds(roff, grows)], out_sems[g])

        n_full = cols // 16
        has_tail = cols % 16 != 0

        def compute(g):
            buf = bufs[g]

            @plsc.parallel_loop(I32(0), I32(grows), step=I32(1), unroll=1)
            def _(r):
                for k in range(n_full - 1):
                    buf[r, pl.ds(16 * k, 16)] = _hash_i32(
                        buf[r, pl.ds(16 * k, 16)])
                o = 16 * (n_full - 1)
                v_last = _hash_i32(buf[r, pl.ds(o, 16)])
                if has_tail:
                    v_tail = _hash_i32(buf[r, pl.ds(cols - 16, 16)])
                buf[r, pl.ds(o, 16)] = v_last
                if has_tail:
                    buf[r, pl.ds(cols - 16, 16)] = v_tail

        hin = [issue_in(0), issue_in(1)]
        hout = []
        for g in range(groups):
            hin[g].wait()
            if g + 2 < groups:
                hin.append(issue_in(g + 2))
            compute(g)
            hout.append(issue_out(g))
        for h in hout:
            h.wait()

    return sc_hash


@jax.jit
def kernel(raw_ids):
    rows, cols = raw_ids.shape
    x = raw_ids.astype(jnp.int32)
    out = _make_sc_call(rows, cols)(x)
    return out.astype(jnp.int64)
